# baseline (speedup 1.0000x reference)
import functools

import jax
import jax.numpy as jnp
from jax import lax
from jax.experimental import pallas as pl
from jax.experimental.pallas import tpu as pltpu
from jax.experimental.pallas import tpu_sc as plsc

_N = 16384
_K = 50
_EMB = 64

_NC = 2
_NS = 16
_NW = _NC * _NS
_ROWS_W = _N // _NW
_E_W = _ROWS_W * _K
_M_ROWS = 8
_M_E = _M_ROWS * _K
_N_M = _ROWS_W // _M_ROWS
_L = 16

_TC_B = 25600
_NB = _N * _K // _TC_B


def _bucket(d):
    b = jnp.minimum(jnp.maximum(d - 1, 0), 1)
    for thr in (2, 3, 4, 7, 15, 31, 63):
        b = b + jnp.minimum(jnp.maximum(d - thr, 0), 1)
    return b


def _sc_index_body(top_hbm, idx_hbm, t2_v, b_v):
    wid = lax.axis_index("s") * _NC + lax.axis_index("c")
    r0 = wid * _ROWS_W
    pltpu.sync_copy(top_hbm.at[pl.ds(r0, _ROWS_W), :], t2_v)

    def macro(m, carry):
        for r in range(_M_ROWS):
            i = r0 + m * _M_ROWS + r
            for off in (0, 16, 32, _K - _L):
                t = t2_v[m * _M_ROWS + r, pl.ds(off, _L)]
                d = jnp.maximum(i - t, 1)
                b_v[pl.ds((m * _M_ROWS + r) * _K + off, _L)] = _bucket(d)
        return carry

    lax.fori_loop(0, _N_M, macro, 0)
    pltpu.sync_copy(b_v, idx_hbm.at[pl.ds(r0 * _K, _E_W)])


def _tc_expand_body(idx_ref, tab_ref, out_ref):
    b = idx_ref[0, 0, :]
    oh = (b[:, None] == lax.broadcasted_iota(jnp.int32, (1, 16), 1))
    out_ref[...] = jnp.dot(oh.astype(jnp.float32), tab_ref[...],
                           preferred_element_type=jnp.float32)


@functools.partial(jax.jit)
def _run(top_2d, distance_emb):
    mesh = plsc.VectorSubcoreMesh(core_axis_name="c", subcore_axis_name="s")
    sc_index = pl.kernel(
        _sc_index_body,
        mesh=mesh,
        compiler_params=pltpu.CompilerParams(use_tc_tiling_on_sc=True),
        out_type=jax.ShapeDtypeStruct((_N * _K,), jnp.int32),
        scratch_types=[
            pltpu.VMEM((_ROWS_W, _K), jnp.int32),
            pltpu.VMEM((_E_W,), jnp.int32),
        ],
    )
    idx = sc_index(top_2d)
    tab16 = jnp.zeros((16, _EMB), jnp.float32).at[:9].set(distance_emb)
    out = pl.pallas_call(
        _tc_expand_body,
        grid=(_NB,),
        in_specs=[
            pl.BlockSpec((1, 1, _TC_B), lambda i: (i, 0, 0)),
            pl.BlockSpec((16, _EMB), lambda i: (0, 0)),
        ],
        out_specs=pl.BlockSpec((_TC_B, _EMB), lambda i: (i, 0)),
        out_shape=jax.ShapeDtypeStruct((_N * _K, _EMB), jnp.float32),
    )(idx.reshape(_NB, 1, _TC_B), tab16)
    return out


def kernel(top_indices, distance_emb):
    out = _run(top_indices, distance_emb)
    return out.reshape(_N, _K, _EMB)
